# Initial kernel scaffold; baseline (speedup 1.0000x reference)
import jax
import jax.numpy as jnp
from jax.experimental import pallas as pl


def kernel(x, edge_index, Wp, bp, A_W, A_b, P_W):
    raise NotImplementedError("write your pallas kernel here")


# speedup vs baseline: 3.4299x; 3.4299x over previous
import functools

import jax
import jax.numpy as jnp
from jax import lax
from jax.experimental import pallas as pl
from jax.experimental.pallas import tpu as pltpu
from jax.experimental.pallas import tpu_sc as plsc

N_NODES = 10000
N_PAD = 10240
N_EDGES = 160000
E_PAD = 163840
D = 256
H = 128
NUM_LAYERS = 4
NC, NS = 2, 16
BATCH = 128
DUMMY_COL = 10200

BLK = 512
GRID = N_PAD // BLK


def _count_body(rowi, coli, out, cnt_a, cnt_b, idxr, idxc, csp_a, csp_b,
                red_buf, res):
    cid = lax.axis_index("c")
    sid = lax.axis_index("s")
    wid = cid * NS + sid

    z16 = jnp.zeros((16,), jnp.float32)

    def zero_body(i, _):
        cnt_a[pl.ds(i * 16, 16)] = z16
        cnt_b[pl.ds(i * 16, 16)] = z16
        return 0
    lax.fori_loop(0, N_PAD // 16, zero_body, 0)

    ones = jnp.ones((16,), jnp.float32)
    e_per_tile = E_PAD // (NC * NS)

    def batch_body(b, _):
        base = wid * e_per_tile + b * BATCH
        pltpu.sync_copy(rowi.at[pl.ds(base, BATCH)], idxr)
        pltpu.sync_copy(coli.at[pl.ds(base, BATCH)], idxc)
        for j in range(BATCH // 16):
            r = idxr[pl.ds(j * 16, 16)]
            c = idxc[pl.ds(j * 16, 16)]
            plsc.addupdate_scatter(cnt_a, [c], ones)
            plsc.addupdate_scatter(cnt_b, [c], ones, mask=r == c)
        return 0
    lax.fori_loop(0, e_per_tile // BATCH, batch_body, 0)

    pltpu.sync_copy(cnt_a, csp_a.at[sid])
    pltpu.sync_copy(cnt_b, csp_b.at[sid])
    plsc.subcore_barrier()

    cols = N_PAD // NS
    cbase = sid * cols
    for which, csp in ((0, csp_a), (1, csp_b)):
        pltpu.sync_copy(csp.at[:, pl.ds(cbase, cols)], red_buf)
        def red_body(ch, _):
            acc = red_buf[0, pl.ds(ch * 16, 16)]
            for i in range(1, NS):
                acc = acc + red_buf[i, pl.ds(ch * 16, 16)]
            res[pl.ds(ch * 16, 16)] = acc
            return 0
        lax.fori_loop(0, cols // 16, red_body, 0)
        pltpu.sync_copy(res, out.at[pl.ds(which * 2 * N_PAD + cid * N_PAD
                                          + cbase, cols)])


def _count_call(rowi, coli):
    mesh = plsc.VectorSubcoreMesh(core_axis_name="c", subcore_axis_name="s")
    f = pl.kernel(
        _count_body,
        out_type=jax.ShapeDtypeStruct((4 * N_PAD,), jnp.float32),
        mesh=mesh,
        scratch_types=[
            pltpu.VMEM((N_PAD,), jnp.float32),
            pltpu.VMEM((N_PAD,), jnp.float32),
            pltpu.VMEM((BATCH,), jnp.int32),
            pltpu.VMEM((BATCH,), jnp.int32),
            pltpu.VMEM_SHARED((NS, N_PAD), jnp.float32),
            pltpu.VMEM_SHARED((NS, N_PAD), jnp.float32),
            pltpu.VMEM((NS, N_PAD // NS), jnp.float32),
            pltpu.VMEM((N_PAD // NS,), jnp.float32),
        ],
        compiler_params=pltpu.CompilerParams(needs_layout_passes=False),
    )
    return f(rowi, coli)


def _agg_body(gs0, gs1, rowi, coli, s0, s1, aggs, idxr, idxc, rows, sem):
    cid = lax.axis_index("c")
    sid = lax.axis_index("s")

    z16 = jnp.zeros((16,), jnp.float32)
    def zrow(i, _):
        for j in range(H // 16):
            rows[i, pl.ds(j * 16, 16)] = z16
        return 0
    lax.fori_loop(0, BATCH, zrow, 0)

    rows_per_tile = N_PAD // NS
    def zspmem(k, _):
        pltpu.sync_copy(rows, aggs.at[pl.ds(sid * rows_per_tile + k * BATCH,
                                            BATCH)])
        return 0
    lax.fori_loop(0, rows_per_tile // BATCH, zspmem, 0)
    plsc.subcore_barrier()

    e_per_tile = E_PAD // NS

    def batch_body(b, _):
        base = sid * e_per_tile + b * BATCH
        pltpu.sync_copy(rowi.at[pl.ds(base, BATCH)], idxr)
        pltpu.sync_copy(coli.at[pl.ds(base, BATCH)], idxc)

        @pl.when(cid == 0)
        def _():
            pltpu.async_copy(gs0.at[idxr], rows, sem).wait()

        @pl.when(cid == 1)
        def _():
            pltpu.async_copy(gs1.at[idxr], rows, sem).wait()

        pltpu.sync_copy(rows, aggs.at[idxc], add=True)
        return 0
    lax.fori_loop(0, e_per_tile // BATCH, batch_body, 0)
    plsc.subcore_barrier()

    obase = sid * rows_per_tile

    @pl.when(cid == 0)
    def _():
        pltpu.sync_copy(aggs.at[pl.ds(obase, rows_per_tile)],
                        s0.at[pl.ds(obase, rows_per_tile)])

    @pl.when(cid == 1)
    def _():
        pltpu.sync_copy(aggs.at[pl.ds(obase, rows_per_tile)],
                        s1.at[pl.ds(obase, rows_per_tile)])


def _agg_call(gs0, gs1, rowi, coli):
    mesh = plsc.VectorSubcoreMesh(core_axis_name="c", subcore_axis_name="s")
    f = pl.kernel(
        _agg_body,
        out_type=(jax.ShapeDtypeStruct((N_PAD, H), jnp.float32),
                  jax.ShapeDtypeStruct((N_PAD, H), jnp.float32)),
        mesh=mesh,
        scratch_types=[
            pltpu.VMEM_SHARED((N_PAD, H), jnp.float32),
            pltpu.VMEM((BATCH,), jnp.int32),
            pltpu.VMEM((BATCH,), jnp.int32),
            pltpu.VMEM((BATCH, H), jnp.float32),
            pltpu.SemaphoreType.DMA,
        ],
    )
    return f(gs0, gs1, rowi, coli)


def _gelu(v):
    return 0.5 * v * (1.0 + lax.erf(v * (2.0 ** -0.5)))


def _prep_body(pw_ref, aw_ref, wsym_ref, m_ref):
    P = pw_ref[...]
    A = aw_ref[...]
    ri = lax.broadcasted_iota(jnp.int32, (D, D), 0)
    ci = lax.broadcasted_iota(jnp.int32, (D, D), 1)
    W0u = jnp.where(ci > ri, P[:, :D], 0.0)
    W0 = W0u + W0u.T
    rowsum = jnp.sum(jnp.abs(W0), axis=1, keepdims=True)
    q = P[:, D:D + 1]
    r = P[:, D + 1:D + 2]
    diag = q * rowsum + r
    wsym_ref[...] = W0 + jnp.where(ci == ri, diag, 0.0)
    m_ref[...] = A.T - A


def _prep_call(P_W, A_W):
    return pl.pallas_call(
        _prep_body,
        out_shape=(jax.ShapeDtypeStruct((D, D), jnp.float32),
                   jax.ShapeDtypeStruct((D, D), jnp.float32)),
    )(P_W, A_W)


def _head_body(x_ref, wp_ref, bp_ref, wsym_ref, m_ref, ab_ref,
               ca0_ref, ca1_ref, cb0_ref, cb1_ref,
               h_ref, g_ref, gs0_ref, gs1_ref, u_ref, dis_ref, selfc_ref):
    indeg = ca0_ref[...] + ca1_ref[...]
    slc = cb0_ref[...] + cb1_ref[...]
    deg = indeg - slc
    dis = jnp.where(deg > 0, lax.rsqrt(deg), 0.0)
    dis_ref[...] = dis
    selfc_ref[...] = dis * dis * slc

    x = x_ref[...]
    pre = lax.dot_general(x, wp_ref[...], (((1,), (1,)), ((), ())),
                          preferred_element_type=jnp.float32) + bp_ref[...]
    h = _gelu(pre)
    h_ref[...] = h
    g = jnp.dot(h, wsym_ref[...], preferred_element_type=jnp.float32)
    g_ref[...] = g
    gs = dis * g
    gs0_ref[...] = gs[:, :H]
    gs1_ref[...] = gs[:, H:]
    u_ref[...] = jnp.dot(h, m_ref[...],
                         preferred_element_type=jnp.float32) + ab_ref[...]


def _head_call(x, Wp, bp2, Wsym, M, Ab2, ca0, ca1, cb0, cb1):
    full = pl.BlockSpec((D, D), lambda i: (0, 0))
    vec = pl.BlockSpec((1, D), lambda i: (0, 0))
    nblk = pl.BlockSpec((BLK, D), lambda i: (i, 0))
    hblk = pl.BlockSpec((BLK, H), lambda i: (i, 0))
    cblk = pl.BlockSpec((BLK, 1), lambda i: (i, 0))
    return pl.pallas_call(
        _head_body,
        grid=(GRID,),
        in_specs=[nblk, full, vec, full, full, vec, cblk, cblk, cblk, cblk],
        out_specs=[nblk, nblk, hblk, hblk, nblk, cblk, cblk],
        out_shape=(jax.ShapeDtypeStruct((N_PAD, D), jnp.float32),
                   jax.ShapeDtypeStruct((N_PAD, D), jnp.float32),
                   jax.ShapeDtypeStruct((N_PAD, H), jnp.float32),
                   jax.ShapeDtypeStruct((N_PAD, H), jnp.float32),
                   jax.ShapeDtypeStruct((N_PAD, D), jnp.float32),
                   jax.ShapeDtypeStruct((N_PAD, 1), jnp.float32),
                   jax.ShapeDtypeStruct((N_PAD, 1), jnp.float32)),
    )(x, Wp, bp2, Wsym, M, Ab2, ca0, ca1, cb0, cb1)


def _layer_body(h_ref, g_ref, u_ref, s0_ref, s1_ref, dis_ref, selfc_ref,
                wsym_ref, m_ref, ab_ref,
                hn_ref, gn_ref, gs0_ref, gs1_ref, un_ref):
    dis = dis_ref[...]
    S = jnp.concatenate([s0_ref[...], s1_ref[...]], axis=1)
    agg = dis * S - selfc_ref[...] * g_ref[...]
    hn = h_ref[...] + _gelu(agg - u_ref[...])
    hn_ref[...] = hn
    gn = jnp.dot(hn, wsym_ref[...], preferred_element_type=jnp.float32)
    gn_ref[...] = gn
    gs = dis * gn
    gs0_ref[...] = gs[:, :H]
    gs1_ref[...] = gs[:, H:]
    un_ref[...] = jnp.dot(hn, m_ref[...],
                          preferred_element_type=jnp.float32) + ab_ref[...]


def _layer_call(h, g, u, s0, s1, dis, selfc, Wsym, M, Ab2):
    full = pl.BlockSpec((D, D), lambda i: (0, 0))
    vec = pl.BlockSpec((1, D), lambda i: (0, 0))
    nblk = pl.BlockSpec((BLK, D), lambda i: (i, 0))
    hblk = pl.BlockSpec((BLK, H), lambda i: (i, 0))
    cblk = pl.BlockSpec((BLK, 1), lambda i: (i, 0))
    return pl.pallas_call(
        _layer_body,
        grid=(GRID,),
        in_specs=[nblk, nblk, nblk, hblk, hblk, cblk, cblk, full, full, vec],
        out_specs=[nblk, nblk, hblk, hblk, nblk],
        out_shape=(jax.ShapeDtypeStruct((N_PAD, D), jnp.float32),
                   jax.ShapeDtypeStruct((N_PAD, D), jnp.float32),
                   jax.ShapeDtypeStruct((N_PAD, H), jnp.float32),
                   jax.ShapeDtypeStruct((N_PAD, H), jnp.float32),
                   jax.ShapeDtypeStruct((N_PAD, D), jnp.float32)),
    )(h, g, u, s0, s1, dis, selfc, Wsym, M, Ab2)


def _tail_body(h_ref, g_ref, u_ref, s0_ref, s1_ref, dis_ref, selfc_ref,
               hn_ref):
    S = jnp.concatenate([s0_ref[...], s1_ref[...]], axis=1)
    agg = dis_ref[...] * S - selfc_ref[...] * g_ref[...]
    hn_ref[...] = h_ref[...] + _gelu(agg - u_ref[...])


def _tail_call(h, g, u, s0, s1, dis, selfc):
    nblk = pl.BlockSpec((BLK, D), lambda i: (i, 0))
    hblk = pl.BlockSpec((BLK, H), lambda i: (i, 0))
    cblk = pl.BlockSpec((BLK, 1), lambda i: (i, 0))
    return pl.pallas_call(
        _tail_body,
        grid=(GRID,),
        in_specs=[nblk, nblk, nblk, hblk, hblk, cblk, cblk],
        out_specs=nblk,
        out_shape=jax.ShapeDtypeStruct((N_PAD, D), jnp.float32),
    )(h, g, u, s0, s1, dis, selfc)


def kernel(x, edge_index, Wp, bp, A_W, A_b, P_W):
    row = edge_index[0].astype(jnp.int32)
    col = edge_index[1].astype(jnp.int32)
    npad = E_PAD - N_EDGES
    rowp = jnp.concatenate([row, jnp.zeros((npad,), jnp.int32)])
    colp = jnp.concatenate([col, jnp.full((npad,), DUMMY_COL, jnp.int32)])

    xp = jnp.pad(x, ((0, N_PAD - N_NODES), (0, 0)))
    bp2 = bp.reshape(1, D)
    Ab2 = A_b.reshape(1, D)

    Wsym, M = _prep_call(P_W, A_W)
    cnt = _count_call(rowp, colp)
    ca0 = cnt[0 * N_PAD:1 * N_PAD].reshape(N_PAD, 1)
    ca1 = cnt[1 * N_PAD:2 * N_PAD].reshape(N_PAD, 1)
    cb0 = cnt[2 * N_PAD:3 * N_PAD].reshape(N_PAD, 1)
    cb1 = cnt[3 * N_PAD:4 * N_PAD].reshape(N_PAD, 1)

    h, g, gs0, gs1, u, dis, selfc = _head_call(
        xp, Wp, bp2, Wsym, M, Ab2, ca0, ca1, cb0, cb1)

    for _ in range(NUM_LAYERS - 1):
        s0, s1 = _agg_call(gs0, gs1, rowp, colp)
        h, g, gs0, gs1, u = _layer_call(h, g, u, s0, s1, dis, selfc,
                                        Wsym, M, Ab2)
    s0, s1 = _agg_call(gs0, gs1, rowp, colp)
    h = _tail_call(h, g, u, s0, s1, dis, selfc)
    return h[:N_NODES]


# speedup vs baseline: 4.4331x; 1.2925x over previous
import functools

import jax
import jax.numpy as jnp
from jax import lax
from jax.experimental import pallas as pl
from jax.experimental.pallas import tpu as pltpu
from jax.experimental.pallas import tpu_sc as plsc

N_NODES = 10000
N_PAD = 10240
N_EDGES = 160000
E_PAD = 163840
D = 256
H = 128
NUM_LAYERS = 4
NC, NS = 2, 16
BATCH = 128
DUMMY_COL = 10200

BLK = 512
GRID = N_PAD // BLK


def _count_body(rowi, coli, out, cnt_a, cnt_b, idxr, idxc, csp_a, csp_b,
                red_buf, res):
    cid = lax.axis_index("c")
    sid = lax.axis_index("s")
    wid = cid * NS + sid

    z16 = jnp.zeros((16,), jnp.float32)

    def zero_body(i, _):
        cnt_a[pl.ds(i * 16, 16)] = z16
        cnt_b[pl.ds(i * 16, 16)] = z16
        return 0
    lax.fori_loop(0, N_PAD // 16, zero_body, 0)

    ones = jnp.ones((16,), jnp.float32)
    e_per_tile = E_PAD // (NC * NS)

    def batch_body(b, _):
        base = wid * e_per_tile + b * BATCH
        pltpu.sync_copy(rowi.at[pl.ds(base, BATCH)], idxr)
        pltpu.sync_copy(coli.at[pl.ds(base, BATCH)], idxc)
        for j in range(BATCH // 16):
            r = idxr[pl.ds(j * 16, 16)]
            c = idxc[pl.ds(j * 16, 16)]
            plsc.addupdate_scatter(cnt_a, [c], ones)
            plsc.addupdate_scatter(cnt_b, [c], ones, mask=r == c)
        return 0
    lax.fori_loop(0, e_per_tile // BATCH, batch_body, 0)

    pltpu.sync_copy(cnt_a, csp_a.at[sid])
    pltpu.sync_copy(cnt_b, csp_b.at[sid])
    plsc.subcore_barrier()

    cols = N_PAD // NS
    cbase = sid * cols
    for which, csp in ((0, csp_a), (1, csp_b)):
        pltpu.sync_copy(csp.at[:, pl.ds(cbase, cols)], red_buf)
        def red_body(ch, _):
            acc = red_buf[0, pl.ds(ch * 16, 16)]
            for i in range(1, NS):
                acc = acc + red_buf[i, pl.ds(ch * 16, 16)]
            res[pl.ds(ch * 16, 16)] = acc
            return 0
        lax.fori_loop(0, cols // 16, red_body, 0)
        pltpu.sync_copy(res, out.at[pl.ds(which * 2 * N_PAD + cid * N_PAD
                                          + cbase, cols)])


def _count_call(rowi, coli):
    mesh = plsc.VectorSubcoreMesh(core_axis_name="c", subcore_axis_name="s")
    f = pl.kernel(
        _count_body,
        out_type=jax.ShapeDtypeStruct((4 * N_PAD,), jnp.float32),
        mesh=mesh,
        scratch_types=[
            pltpu.VMEM((N_PAD,), jnp.float32),
            pltpu.VMEM((N_PAD,), jnp.float32),
            pltpu.VMEM((BATCH,), jnp.int32),
            pltpu.VMEM((BATCH,), jnp.int32),
            pltpu.VMEM_SHARED((NS, N_PAD), jnp.float32),
            pltpu.VMEM_SHARED((NS, N_PAD), jnp.float32),
            pltpu.VMEM((NS, N_PAD // NS), jnp.float32),
            pltpu.VMEM((N_PAD // NS,), jnp.float32),
        ],
        compiler_params=pltpu.CompilerParams(needs_layout_passes=False),
    )
    return f(rowi, coli)


NBUF = 4
EB = 80
NBATCH = E_PAD // NS // EB


def _agg_body(gs0, gs1, eidx, s0, s1, aggs, idxb, rows, g0, g1, g2, g3):
    cid = lax.axis_index("c")
    sid = lax.axis_index("s")
    gsems = (g0, g1, g2, g3)

    z16 = jnp.zeros((16,), jnp.float32)
    def zrow(i, _):
        for j in range(H // 16):
            rows[0, i, pl.ds(j * 16, 16)] = z16
        return 0
    lax.fori_loop(0, EB, zrow, 0)

    rows_per_tile = N_PAD // NS
    def zspmem(k, _):
        pltpu.sync_copy(rows.at[0, pl.ds(0, EB)],
                        aggs.at[pl.ds(sid * rows_per_tile + k * EB, EB)])
        return 0
    lax.fori_loop(0, rows_per_tile // EB, zspmem, 0)
    plsc.subcore_barrier()

    def gather_start(b, s):
        pltpu.sync_copy(eidx.at[sid, b], idxb.at[s])

        @pl.when(cid == 0)
        def _():
            pltpu.async_copy(gs0.at[idxb.at[s, 0]], rows.at[s], gsems[s])

        @pl.when(cid == 1)
        def _():
            pltpu.async_copy(gs1.at[idxb.at[s, 0]], rows.at[s], gsems[s])

    for s in range(NBUF):
        gather_start(s, s)

    def outer(o, _):
        for s in range(NBUF):
            b = o * NBUF + s
            pltpu.make_async_copy(gs0.at[pl.ds(0, EB)], rows.at[s],
                                  gsems[s]).wait()
            pltpu.sync_copy(rows.at[s], aggs.at[idxb.at[s, 1]], add=True)

            @pl.when(o < NBATCH // NBUF - 1)
            def _():
                gather_start(b + NBUF, s)
        return 0
    lax.fori_loop(0, NBATCH // NBUF, outer, 0)
    plsc.subcore_barrier()

    obase = sid * rows_per_tile

    @pl.when(cid == 0)
    def _():
        pltpu.sync_copy(aggs.at[pl.ds(obase, rows_per_tile)],
                        s0.at[pl.ds(obase, rows_per_tile)])

    @pl.when(cid == 1)
    def _():
        pltpu.sync_copy(aggs.at[pl.ds(obase, rows_per_tile)],
                        s1.at[pl.ds(obase, rows_per_tile)])


def _agg_call(gs0, gs1, eidx):
    mesh = plsc.VectorSubcoreMesh(core_axis_name="c", subcore_axis_name="s")
    f = pl.kernel(
        _agg_body,
        out_type=(jax.ShapeDtypeStruct((N_PAD, H), jnp.float32),
                  jax.ShapeDtypeStruct((N_PAD, H), jnp.float32)),
        mesh=mesh,
        scratch_types=[
            pltpu.VMEM_SHARED((N_PAD, H), jnp.float32),
            pltpu.VMEM((NBUF, 2, EB), jnp.int32),
            pltpu.VMEM((NBUF, EB, H), jnp.float32),
            pltpu.SemaphoreType.DMA,
            pltpu.SemaphoreType.DMA,
            pltpu.SemaphoreType.DMA,
            pltpu.SemaphoreType.DMA,
        ],
    )
    return f(gs0, gs1, eidx)


def _gelu(v):
    return 0.5 * v * (1.0 + lax.erf(v * (2.0 ** -0.5)))


def _prep_body(pw_ref, aw_ref, wsym_ref, m_ref):
    P = pw_ref[...]
    A = aw_ref[...]
    ri = lax.broadcasted_iota(jnp.int32, (D, D), 0)
    ci = lax.broadcasted_iota(jnp.int32, (D, D), 1)
    W0u = jnp.where(ci > ri, P[:, :D], 0.0)
    W0 = W0u + W0u.T
    rowsum = jnp.sum(jnp.abs(W0), axis=1, keepdims=True)
    q = P[:, D:D + 1]
    r = P[:, D + 1:D + 2]
    diag = q * rowsum + r
    wsym_ref[...] = W0 + jnp.where(ci == ri, diag, 0.0)
    m_ref[...] = A.T - A


def _prep_call(P_W, A_W):
    return pl.pallas_call(
        _prep_body,
        out_shape=(jax.ShapeDtypeStruct((D, D), jnp.float32),
                   jax.ShapeDtypeStruct((D, D), jnp.float32)),
    )(P_W, A_W)


def _head_body(x_ref, wp_ref, bp_ref, wsym_ref, m_ref, ab_ref,
               ca0_ref, ca1_ref, cb0_ref, cb1_ref,
               h_ref, g_ref, gs0_ref, gs1_ref, u_ref, dis_ref, selfc_ref):
    indeg = ca0_ref[...] + ca1_ref[...]
    slc = cb0_ref[...] + cb1_ref[...]
    deg = indeg - slc
    dis = jnp.where(deg > 0, lax.rsqrt(deg), 0.0)
    dis_ref[...] = dis
    selfc_ref[...] = dis * dis * slc

    x = x_ref[...]
    pre = lax.dot_general(x, wp_ref[...], (((1,), (1,)), ((), ())),
                          preferred_element_type=jnp.float32) + bp_ref[...]
    h = _gelu(pre)
    h_ref[...] = h
    g = jnp.dot(h, wsym_ref[...], preferred_element_type=jnp.float32)
    g_ref[...] = g
    gs = dis * g
    gs0_ref[...] = gs[:, :H]
    gs1_ref[...] = gs[:, H:]
    u_ref[...] = jnp.dot(h, m_ref[...],
                         preferred_element_type=jnp.float32) + ab_ref[...]


def _head_call(x, Wp, bp2, Wsym, M, Ab2, ca0, ca1, cb0, cb1):
    full = pl.BlockSpec((D, D), lambda i: (0, 0))
    vec = pl.BlockSpec((1, D), lambda i: (0, 0))
    nblk = pl.BlockSpec((BLK, D), lambda i: (i, 0))
    hblk = pl.BlockSpec((BLK, H), lambda i: (i, 0))
    cblk = pl.BlockSpec((BLK, 1), lambda i: (i, 0))
    return pl.pallas_call(
        _head_body,
        grid=(GRID,),
        in_specs=[nblk, full, vec, full, full, vec, cblk, cblk, cblk, cblk],
        out_specs=[nblk, nblk, hblk, hblk, nblk, cblk, cblk],
        out_shape=(jax.ShapeDtypeStruct((N_PAD, D), jnp.float32),
                   jax.ShapeDtypeStruct((N_PAD, D), jnp.float32),
                   jax.ShapeDtypeStruct((N_PAD, H), jnp.float32),
                   jax.ShapeDtypeStruct((N_PAD, H), jnp.float32),
                   jax.ShapeDtypeStruct((N_PAD, D), jnp.float32),
                   jax.ShapeDtypeStruct((N_PAD, 1), jnp.float32),
                   jax.ShapeDtypeStruct((N_PAD, 1), jnp.float32)),
    )(x, Wp, bp2, Wsym, M, Ab2, ca0, ca1, cb0, cb1)


def _layer_body(h_ref, g_ref, u_ref, s0_ref, s1_ref, dis_ref, selfc_ref,
                wsym_ref, m_ref, ab_ref,
                hn_ref, gn_ref, gs0_ref, gs1_ref, un_ref):
    dis = dis_ref[...]
    S = jnp.concatenate([s0_ref[...], s1_ref[...]], axis=1)
    agg = dis * S - selfc_ref[...] * g_ref[...]
    hn = h_ref[...] + _gelu(agg - u_ref[...])
    hn_ref[...] = hn
    gn = jnp.dot(hn, wsym_ref[...], preferred_element_type=jnp.float32)
    gn_ref[...] = gn
    gs = dis * gn
    gs0_ref[...] = gs[:, :H]
    gs1_ref[...] = gs[:, H:]
    un_ref[...] = jnp.dot(hn, m_ref[...],
                          preferred_element_type=jnp.float32) + ab_ref[...]


def _layer_call(h, g, u, s0, s1, dis, selfc, Wsym, M, Ab2):
    full = pl.BlockSpec((D, D), lambda i: (0, 0))
    vec = pl.BlockSpec((1, D), lambda i: (0, 0))
    nblk = pl.BlockSpec((BLK, D), lambda i: (i, 0))
    hblk = pl.BlockSpec((BLK, H), lambda i: (i, 0))
    cblk = pl.BlockSpec((BLK, 1), lambda i: (i, 0))
    return pl.pallas_call(
        _layer_body,
        grid=(GRID,),
        in_specs=[nblk, nblk, nblk, hblk, hblk, cblk, cblk, full, full, vec],
        out_specs=[nblk, nblk, hblk, hblk, nblk],
        out_shape=(jax.ShapeDtypeStruct((N_PAD, D), jnp.float32),
                   jax.ShapeDtypeStruct((N_PAD, D), jnp.float32),
                   jax.ShapeDtypeStruct((N_PAD, H), jnp.float32),
                   jax.ShapeDtypeStruct((N_PAD, H), jnp.float32),
                   jax.ShapeDtypeStruct((N_PAD, D), jnp.float32)),
    )(h, g, u, s0, s1, dis, selfc, Wsym, M, Ab2)


def _tail_body(h_ref, g_ref, u_ref, s0_ref, s1_ref, dis_ref, selfc_ref,
               hn_ref):
    S = jnp.concatenate([s0_ref[...], s1_ref[...]], axis=1)
    agg = dis_ref[...] * S - selfc_ref[...] * g_ref[...]
    hn_ref[...] = h_ref[...] + _gelu(agg - u_ref[...])


def _tail_call(h, g, u, s0, s1, dis, selfc):
    nblk = pl.BlockSpec((BLK, D), lambda i: (i, 0))
    hblk = pl.BlockSpec((BLK, H), lambda i: (i, 0))
    cblk = pl.BlockSpec((BLK, 1), lambda i: (i, 0))
    return pl.pallas_call(
        _tail_body,
        grid=(GRID,),
        in_specs=[nblk, nblk, nblk, hblk, hblk, cblk, cblk],
        out_specs=nblk,
        out_shape=jax.ShapeDtypeStruct((N_PAD, D), jnp.float32),
    )(h, g, u, s0, s1, dis, selfc)


def kernel(x, edge_index, Wp, bp, A_W, A_b, P_W):
    row = edge_index[0].astype(jnp.int32)
    col = edge_index[1].astype(jnp.int32)
    npad = E_PAD - N_EDGES
    rowp = jnp.concatenate([row, jnp.zeros((npad,), jnp.int32)])
    colp = jnp.concatenate([col, jnp.full((npad,), DUMMY_COL, jnp.int32)])
    eidx = jnp.stack([rowp.reshape(NS, NBATCH, EB),
                      colp.reshape(NS, NBATCH, EB)], axis=2)

    xp = jnp.pad(x, ((0, N_PAD - N_NODES), (0, 0)))
    bp2 = bp.reshape(1, D)
    Ab2 = A_b.reshape(1, D)

    Wsym, M = _prep_call(P_W, A_W)
    cnt = _count_call(rowp, colp)
    ca0 = cnt[0 * N_PAD:1 * N_PAD].reshape(N_PAD, 1)
    ca1 = cnt[1 * N_PAD:2 * N_PAD].reshape(N_PAD, 1)
    cb0 = cnt[2 * N_PAD:3 * N_PAD].reshape(N_PAD, 1)
    cb1 = cnt[3 * N_PAD:4 * N_PAD].reshape(N_PAD, 1)

    h, g, gs0, gs1, u, dis, selfc = _head_call(
        xp, Wp, bp2, Wsym, M, Ab2, ca0, ca1, cb0, cb1)

    for _ in range(NUM_LAYERS - 1):
        s0, s1 = _agg_call(gs0, gs1, eidx)
        h, g, gs0, gs1, u = _layer_call(h, g, u, s0, s1, dis, selfc,
                                        Wsym, M, Ab2)
    s0, s1 = _agg_call(gs0, gs1, eidx)
    h = _tail_call(h, g, u, s0, s1, dis, selfc)
    return h[:N_NODES]


# speedup vs baseline: 4.8912x; 1.1033x over previous
import functools

import jax
import jax.numpy as jnp
from jax import lax
from jax.experimental import pallas as pl
from jax.experimental.pallas import tpu as pltpu
from jax.experimental.pallas import tpu_sc as plsc

N_NODES = 10000
N_PAD = 10240
N_EDGES = 160000
E_PAD = 163840
D = 256
H = 128
NUM_LAYERS = 4
NC, NS = 2, 16
BATCH = 128
DUMMY_COL = 10200

BLK = 512
GRID = N_PAD // BLK


def _count_body(rowi, coli, out, cnt_a, cnt_b, idxr, idxc, csp_a, csp_b,
                red_buf, res):
    cid = lax.axis_index("c")
    sid = lax.axis_index("s")
    wid = cid * NS + sid

    z16 = jnp.zeros((16,), jnp.float32)

    def zero_body(i, _):
        cnt_a[pl.ds(i * 16, 16)] = z16
        cnt_b[pl.ds(i * 16, 16)] = z16
        return 0
    lax.fori_loop(0, N_PAD // 16, zero_body, 0)

    ones = jnp.ones((16,), jnp.float32)
    e_per_tile = E_PAD // (NC * NS)

    def batch_body(b, _):
        base = wid * e_per_tile + b * BATCH
        pltpu.sync_copy(rowi.at[pl.ds(base, BATCH)], idxr)
        pltpu.sync_copy(coli.at[pl.ds(base, BATCH)], idxc)
        for j in range(BATCH // 16):
            r = idxr[pl.ds(j * 16, 16)]
            c = idxc[pl.ds(j * 16, 16)]
            plsc.addupdate_scatter(cnt_a, [c], ones)
            plsc.addupdate_scatter(cnt_b, [c], ones, mask=r == c)
        return 0
    lax.fori_loop(0, e_per_tile // BATCH, batch_body, 0)

    pltpu.sync_copy(cnt_a, csp_a.at[sid])
    pltpu.sync_copy(cnt_b, csp_b.at[sid])
    plsc.subcore_barrier()

    cols = N_PAD // NS
    cbase = sid * cols
    for which, csp in ((0, csp_a), (1, csp_b)):
        pltpu.sync_copy(csp.at[:, pl.ds(cbase, cols)], red_buf)
        def red_body(ch, _):
            acc = red_buf[0, pl.ds(ch * 16, 16)]
            for i in range(1, NS):
                acc = acc + red_buf[i, pl.ds(ch * 16, 16)]
            res[pl.ds(ch * 16, 16)] = acc
            return 0
        lax.fori_loop(0, cols // 16, red_body, 0)
        pltpu.sync_copy(res, out.at[pl.ds(which * 2 * N_PAD + cid * N_PAD
                                          + cbase, cols)])


def _count_call(rowi, coli):
    mesh = plsc.VectorSubcoreMesh(core_axis_name="c", subcore_axis_name="s")
    f = pl.kernel(
        _count_body,
        out_type=jax.ShapeDtypeStruct((4 * N_PAD,), jnp.float32),
        mesh=mesh,
        scratch_types=[
            pltpu.VMEM((N_PAD,), jnp.float32),
            pltpu.VMEM((N_PAD,), jnp.float32),
            pltpu.VMEM((BATCH,), jnp.int32),
            pltpu.VMEM((BATCH,), jnp.int32),
            pltpu.VMEM_SHARED((NS, N_PAD), jnp.float32),
            pltpu.VMEM_SHARED((NS, N_PAD), jnp.float32),
            pltpu.VMEM((NS, N_PAD // NS), jnp.float32),
            pltpu.VMEM((N_PAD // NS,), jnp.float32),
        ],
        compiler_params=pltpu.CompilerParams(needs_layout_passes=False),
    )
    return f(rowi, coli)


NBUF = 4
EB = 64
NBATCH = E_PAD // NS // EB


def _agg_body(gs0, gs1, eidx, s0, s1, aggs, pbuf, idxu, rows, g0, g1, g2, g3):
    cid = lax.axis_index("c")
    sid = lax.axis_index("s")
    gsems = (g0, g1, g2, g3)

    pidx = pltpu.async_copy(eidx.at[sid], pbuf, g0)

    z16 = jnp.zeros((16,), jnp.float32)
    def zrow(i, _):
        for j in range(H // 16):
            rows[0, i, pl.ds(j * 16, 16)] = z16
        return 0
    lax.fori_loop(0, EB, zrow, 0)

    rows_per_tile = N_PAD // NS
    def zspmem(k, _):
        pltpu.sync_copy(rows.at[0],
                        aggs.at[pl.ds(sid * rows_per_tile + k * EB, EB)])
        return 0
    lax.fori_loop(0, rows_per_tile // EB, zspmem, 0)
    pidx.wait()
    plsc.subcore_barrier()

    def gather_start(b, s):
        for j in range(EB // 16):
            p = pbuf[b // 2, pl.ds((b % 2) * EB + j * 16, 16)]
            idxu[s, 0, pl.ds(j * 16, 16)] = p & 0xFFFF
            idxu[s, 1, pl.ds(j * 16, 16)] = p >> 16

        @pl.when(cid == 0)
        def _():
            pltpu.async_copy(gs0.at[idxu.at[s, 0]], rows.at[s], gsems[s])

        @pl.when(cid == 1)
        def _():
            pltpu.async_copy(gs1.at[idxu.at[s, 0]], rows.at[s], gsems[s])

    for s in range(NBUF):
        gather_start(s, s)

    def outer(o, _):
        for s in range(NBUF):
            b = o * NBUF + s
            pltpu.make_async_copy(gs0.at[pl.ds(0, EB)], rows.at[s],
                                  gsems[s]).wait()
            pltpu.sync_copy(rows.at[s], aggs.at[idxu.at[s, 1]], add=True)

            @pl.when(o < NBATCH // NBUF - 1)
            def _():
                gather_start(b + NBUF, s)
        return 0
    lax.fori_loop(0, NBATCH // NBUF, outer, 0)
    plsc.subcore_barrier()

    obase = sid * rows_per_tile

    @pl.when(cid == 0)
    def _():
        pltpu.sync_copy(aggs.at[pl.ds(obase, rows_per_tile)],
                        s0.at[pl.ds(obase, rows_per_tile)])

    @pl.when(cid == 1)
    def _():
        pltpu.sync_copy(aggs.at[pl.ds(obase, rows_per_tile)],
                        s1.at[pl.ds(obase, rows_per_tile)])


def _agg_call(gs0, gs1, eidx):
    mesh = plsc.VectorSubcoreMesh(core_axis_name="c", subcore_axis_name="s")
    f = pl.kernel(
        _agg_body,
        out_type=(jax.ShapeDtypeStruct((N_PAD, H), jnp.float32),
                  jax.ShapeDtypeStruct((N_PAD, H), jnp.float32)),
        mesh=mesh,
        scratch_types=[
            pltpu.VMEM_SHARED((N_PAD, H), jnp.float32),
            pltpu.VMEM((NBATCH // 2, 2 * EB), jnp.int32),
            pltpu.VMEM((NBUF, 2, EB), jnp.int32),
            pltpu.VMEM((NBUF, EB, H), jnp.float32),
            pltpu.SemaphoreType.DMA,
            pltpu.SemaphoreType.DMA,
            pltpu.SemaphoreType.DMA,
            pltpu.SemaphoreType.DMA,
        ],
    )
    return f(gs0, gs1, eidx)


def _gelu(v):
    return 0.5 * v * (1.0 + lax.erf(v * (2.0 ** -0.5)))


def _prep_body(pw_ref, aw_ref, wsym_ref, m_ref):
    P = pw_ref[...]
    A = aw_ref[...]
    ri = lax.broadcasted_iota(jnp.int32, (D, D), 0)
    ci = lax.broadcasted_iota(jnp.int32, (D, D), 1)
    W0u = jnp.where(ci > ri, P[:, :D], 0.0)
    W0 = W0u + W0u.T
    rowsum = jnp.sum(jnp.abs(W0), axis=1, keepdims=True)
    q = P[:, D:D + 1]
    r = P[:, D + 1:D + 2]
    diag = q * rowsum + r
    wsym_ref[...] = W0 + jnp.where(ci == ri, diag, 0.0)
    m_ref[...] = A.T - A


def _prep_call(P_W, A_W):
    return pl.pallas_call(
        _prep_body,
        out_shape=(jax.ShapeDtypeStruct((D, D), jnp.float32),
                   jax.ShapeDtypeStruct((D, D), jnp.float32)),
    )(P_W, A_W)


def _head_body(x_ref, wp_ref, bp_ref, wsym_ref, m_ref, ab_ref,
               ca0_ref, ca1_ref, cb0_ref, cb1_ref,
               h_ref, g_ref, gs0_ref, gs1_ref, u_ref, dis_ref, selfc_ref):
    indeg = ca0_ref[...] + ca1_ref[...]
    slc = cb0_ref[...] + cb1_ref[...]
    deg = indeg - slc
    dis = jnp.where(deg > 0, lax.rsqrt(deg), 0.0)
    dis_ref[...] = dis
    selfc_ref[...] = dis * dis * slc

    x = x_ref[...]
    pre = lax.dot_general(x, wp_ref[...], (((1,), (1,)), ((), ())),
                          preferred_element_type=jnp.float32) + bp_ref[...]
    h = _gelu(pre)
    h_ref[...] = h
    g = jnp.dot(h, wsym_ref[...], preferred_element_type=jnp.float32)
    g_ref[...] = g
    gs = dis * g
    gs0_ref[...] = gs[:, :H]
    gs1_ref[...] = gs[:, H:]
    u_ref[...] = jnp.dot(h, m_ref[...],
                         preferred_element_type=jnp.float32) + ab_ref[...]


def _head_call(x, Wp, bp2, Wsym, M, Ab2, ca0, ca1, cb0, cb1):
    full = pl.BlockSpec((D, D), lambda i: (0, 0))
    vec = pl.BlockSpec((1, D), lambda i: (0, 0))
    nblk = pl.BlockSpec((BLK, D), lambda i: (i, 0))
    hblk = pl.BlockSpec((BLK, H), lambda i: (i, 0))
    cblk = pl.BlockSpec((BLK, 1), lambda i: (i, 0))
    return pl.pallas_call(
        _head_body,
        grid=(GRID,),
        in_specs=[nblk, full, vec, full, full, vec, cblk, cblk, cblk, cblk],
        out_specs=[nblk, nblk, hblk, hblk, nblk, cblk, cblk],
        out_shape=(jax.ShapeDtypeStruct((N_PAD, D), jnp.float32),
                   jax.ShapeDtypeStruct((N_PAD, D), jnp.float32),
                   jax.ShapeDtypeStruct((N_PAD, H), jnp.float32),
                   jax.ShapeDtypeStruct((N_PAD, H), jnp.float32),
                   jax.ShapeDtypeStruct((N_PAD, D), jnp.float32),
                   jax.ShapeDtypeStruct((N_PAD, 1), jnp.float32),
                   jax.ShapeDtypeStruct((N_PAD, 1), jnp.float32)),
    )(x, Wp, bp2, Wsym, M, Ab2, ca0, ca1, cb0, cb1)


def _layer_body(h_ref, g_ref, u_ref, s0_ref, s1_ref, dis_ref, selfc_ref,
                wsym_ref, m_ref, ab_ref,
                hn_ref, gn_ref, gs0_ref, gs1_ref, un_ref):
    dis = dis_ref[...]
    S = jnp.concatenate([s0_ref[...], s1_ref[...]], axis=1)
    agg = dis * S - selfc_ref[...] * g_ref[...]
    hn = h_ref[...] + _gelu(agg - u_ref[...])
    hn_ref[...] = hn
    gn = jnp.dot(hn, wsym_ref[...], preferred_element_type=jnp.float32)
    gn_ref[...] = gn
    gs = dis * gn
    gs0_ref[...] = gs[:, :H]
    gs1_ref[...] = gs[:, H:]
    un_ref[...] = jnp.dot(hn, m_ref[...],
                          preferred_element_type=jnp.float32) + ab_ref[...]


def _layer_call(h, g, u, s0, s1, dis, selfc, Wsym, M, Ab2):
    full = pl.BlockSpec((D, D), lambda i: (0, 0))
    vec = pl.BlockSpec((1, D), lambda i: (0, 0))
    nblk = pl.BlockSpec((BLK, D), lambda i: (i, 0))
    hblk = pl.BlockSpec((BLK, H), lambda i: (i, 0))
    cblk = pl.BlockSpec((BLK, 1), lambda i: (i, 0))
    return pl.pallas_call(
        _layer_body,
        grid=(GRID,),
        in_specs=[nblk, nblk, nblk, hblk, hblk, cblk, cblk, full, full, vec],
        out_specs=[nblk, nblk, hblk, hblk, nblk],
        out_shape=(jax.ShapeDtypeStruct((N_PAD, D), jnp.float32),
                   jax.ShapeDtypeStruct((N_PAD, D), jnp.float32),
                   jax.ShapeDtypeStruct((N_PAD, H), jnp.float32),
                   jax.ShapeDtypeStruct((N_PAD, H), jnp.float32),
                   jax.ShapeDtypeStruct((N_PAD, D), jnp.float32)),
    )(h, g, u, s0, s1, dis, selfc, Wsym, M, Ab2)


def _tail_body(h_ref, g_ref, u_ref, s0_ref, s1_ref, dis_ref, selfc_ref,
               hn_ref):
    S = jnp.concatenate([s0_ref[...], s1_ref[...]], axis=1)
    agg = dis_ref[...] * S - selfc_ref[...] * g_ref[...]
    hn_ref[...] = h_ref[...] + _gelu(agg - u_ref[...])


def _tail_call(h, g, u, s0, s1, dis, selfc):
    nblk = pl.BlockSpec((BLK, D), lambda i: (i, 0))
    hblk = pl.BlockSpec((BLK, H), lambda i: (i, 0))
    cblk = pl.BlockSpec((BLK, 1), lambda i: (i, 0))
    return pl.pallas_call(
        _tail_body,
        grid=(GRID,),
        in_specs=[nblk, nblk, nblk, hblk, hblk, cblk, cblk],
        out_specs=nblk,
        out_shape=jax.ShapeDtypeStruct((N_PAD, D), jnp.float32),
    )(h, g, u, s0, s1, dis, selfc)


def kernel(x, edge_index, Wp, bp, A_W, A_b, P_W):
    row = edge_index[0].astype(jnp.int32)
    col = edge_index[1].astype(jnp.int32)
    npad = E_PAD - N_EDGES
    rowp = jnp.concatenate([row, jnp.zeros((npad,), jnp.int32)])
    colp = jnp.concatenate([col, jnp.full((npad,), DUMMY_COL, jnp.int32)])
    eidx = (rowp | (colp << 16)).reshape(NS, NBATCH // 2, 2 * EB)

    xp = jnp.pad(x, ((0, N_PAD - N_NODES), (0, 0)))
    bp2 = bp.reshape(1, D)
    Ab2 = A_b.reshape(1, D)

    Wsym, M = _prep_call(P_W, A_W)
    cnt = _count_call(rowp, colp)
    ca0 = cnt[0 * N_PAD:1 * N_PAD].reshape(N_PAD, 1)
    ca1 = cnt[1 * N_PAD:2 * N_PAD].reshape(N_PAD, 1)
    cb0 = cnt[2 * N_PAD:3 * N_PAD].reshape(N_PAD, 1)
    cb1 = cnt[3 * N_PAD:4 * N_PAD].reshape(N_PAD, 1)

    h, g, gs0, gs1, u, dis, selfc = _head_call(
        xp, Wp, bp2, Wsym, M, Ab2, ca0, ca1, cb0, cb1)

    for _ in range(NUM_LAYERS - 1):
        s0, s1 = _agg_call(gs0, gs1, eidx)
        h, g, gs0, gs1, u = _layer_call(h, g, u, s0, s1, dis, selfc,
                                        Wsym, M, Ab2)
    s0, s1 = _agg_call(gs0, gs1, eidx)
    h = _tail_call(h, g, u, s0, s1, dis, selfc)
    return h[:N_NODES]
